# Initial kernel scaffold; baseline (speedup 1.0000x reference)
import jax
import jax.numpy as jnp
from jax.experimental import pallas as pl


def kernel(parent_feature, gt_children_code, gt_num_code, W_parent, b_parent, W_exists, b_exists, W_sem, b_sem, W_edge_latent, b_edge_latent, W_edge_exists, b_edge_exists, W_node_edge, b_node_edge, W_child, b_child, W_child2, b_child2):
    raise NotImplementedError("write your pallas kernel here")


# speedup vs baseline: 1.5126x; 1.5126x over previous
import functools

import jax
import jax.numpy as jnp
from jax.experimental import pallas as pl

C = 256
H = 128
T = 4
NITER = 3
BI = 32
NBLK = C // BI


def _dot(a, b):
    return jnp.dot(a, b, preferred_element_type=jnp.float32)


def _parent_kernel(feat_ref, wp_ref, bp_ref, out_ref):
    out_ref[...] = jnp.maximum(_dot(feat_ref[...], wp_ref[...]) + bp_ref[...], 0.0)


def _parent(feat, W_parent, b_parent):
    colb = H * BI
    return pl.pallas_call(
        _parent_kernel,
        grid=(NBLK,),
        in_specs=[
            pl.BlockSpec((1, feat.shape[1]), lambda i: (0, 0)),
            pl.BlockSpec((W_parent.shape[0], colb), lambda i: (0, i)),
            pl.BlockSpec((1, colb), lambda i: (0, i)),
        ],
        out_specs=pl.BlockSpec((1, colb), lambda i: (0, i)),
        out_shape=jax.ShapeDtypeStruct((1, C * H), jnp.float32),
    )(feat, W_parent, b_parent.reshape(1, C * H))


def _edges_kernel(cf0_ref, wa_ref, wb_ref, bel_ref, wee_ref, bee_ref,
                  wex_ref, bex_ref, e_ref, ex_ref, any_ref):
    i = pl.program_id(0)
    cf0 = cf0_ref[...]
    cf_blk = cf0_ref[pl.ds(i * BI, BI), :]
    P = _dot(cf_blk, wa_ref[...])
    Q = _dot(cf0, wb_ref[...])
    el = jnp.maximum(P[:, None, :] + Q[None, :, :] + bel_ref[...][None, :, :], 0.0)
    e4 = _dot(el.reshape(BI * C, H), wee_ref[...].T).reshape(BI, C, T) + bee_ref[...][None, :, :]
    e_ref[...] = e4
    ex_full = _dot(cf0, wex_ref[...]) + bex_ref[...]
    ex_blk = _dot(cf_blk, wex_ref[...]) + bex_ref[...]
    ex_ref[...] = ex_blk
    ne = (ex_full > 0.0).astype(jnp.float32)
    ne_i = (ex_blk > 0.0).astype(jnp.float32)
    m = (e4 > 0.0).astype(jnp.float32) * ne_i[:, :, None] * ne.reshape(1, C, 1)
    blk_any = jnp.max(m).reshape(1, 1)

    @pl.when(i == 0)
    def _():
        any_ref[...] = blk_any

    @pl.when(i > 0)
    def _():
        any_ref[...] = jnp.maximum(any_ref[...], blk_any)


def _edges(cf0, Wa, Wb, b_el, W_ee, b_ee, W_exists, b_exists):
    return pl.pallas_call(
        _edges_kernel,
        grid=(NBLK,),
        in_specs=[
            pl.BlockSpec((C, H), lambda i: (0, 0)),
            pl.BlockSpec((H, H), lambda i: (0, 0)),
            pl.BlockSpec((H, H), lambda i: (0, 0)),
            pl.BlockSpec((1, H), lambda i: (0, 0)),
            pl.BlockSpec((T, H), lambda i: (0, 0)),
            pl.BlockSpec((1, T), lambda i: (0, 0)),
            pl.BlockSpec((H, 1), lambda i: (0, 0)),
            pl.BlockSpec((1, 1), lambda i: (0, 0)),
        ],
        out_specs=[
            pl.BlockSpec((BI, C, T), lambda i: (i, 0, 0)),
            pl.BlockSpec((BI, 1), lambda i: (i, 0)),
            pl.BlockSpec((1, 1), lambda i: (0, 0)),
        ],
        out_shape=[
            jax.ShapeDtypeStruct((C, C, T), jnp.float32),
            jax.ShapeDtypeStruct((C, 1), jnp.float32),
            jax.ShapeDtypeStruct((1, 1), jnp.float32),
        ],
    )(cf0, Wa, Wb, b_el.reshape(1, H), W_ee, b_ee.reshape(1, T),
      W_exists, b_exists.reshape(1, 1))


def _iter_kernel(cf_ref, cf0_ref, wa_ref, wb_ref, bel_ref,
                 w1_ref, w2_ref, w3_ref, w4_ref, bk_ref,
                 e_ref, ex_ref, any_ref, out_ref):
    i = pl.program_id(0)
    cf = cf_ref[...]
    cf0 = cf0_ref[...]
    cf_blk = cf_ref[pl.ds(i * BI, BI), :]
    A = _dot(cf_blk, w1_ref[...])
    Bf = _dot(cf, w2_ref[...])
    P = _dot(cf0_ref[pl.ds(i * BI, BI), :], wa_ref[...])
    Q = _dot(cf0, wb_ref[...])
    el = jnp.maximum(P[:, None, :] + Q[None, :, :] + bel_ref[...][None, :, :], 0.0)
    EL = _dot(el.reshape(BI * C, H), w3_ref[...]).reshape(BI, C, H)
    base = A[:, None, :] + Bf[None, :, :] + EL + bk_ref[...][None, :, :]

    ne = (ex_ref[...] > 0.0).astype(jnp.float32)
    ne_i = (ex_ref[pl.ds(i * BI, BI), :] > 0.0).astype(jnp.float32)
    e4 = e_ref[...]
    w4 = w4_ref[...]

    acc = jnp.zeros((BI, H), jnp.float32)
    cnt = jnp.zeros((BI, 1), jnp.float32)
    for t in range(T):
        et = e4[:, :, t]
        mt = (et > 0.0).astype(jnp.float32) * ne_i * ne.reshape(1, C)
        v = jnp.maximum(base + et[:, :, None] * w4[t, :][None, None, :], 0.0)
        acc = acc + jnp.sum(mt[:, :, None] * v, axis=1)
        cnt = cnt + jnp.sum(mt, axis=1, keepdims=True)

    new_blk = acc / jnp.maximum(cnt, 1.0)
    out_ref[...] = jnp.where(any_ref[...] > 0.0, new_blk, cf_blk)


def _mp_iter(cf, cf0, Wa, Wb, b_el, W1, W2, W3, W4, bk, e_logits, ex, has_any):
    return pl.pallas_call(
        _iter_kernel,
        grid=(NBLK,),
        in_specs=[
            pl.BlockSpec((C, H), lambda i: (0, 0)),
            pl.BlockSpec((C, H), lambda i: (0, 0)),
            pl.BlockSpec((H, H), lambda i: (0, 0)),
            pl.BlockSpec((H, H), lambda i: (0, 0)),
            pl.BlockSpec((1, H), lambda i: (0, 0)),
            pl.BlockSpec((H, H), lambda i: (0, 0)),
            pl.BlockSpec((H, H), lambda i: (0, 0)),
            pl.BlockSpec((H, H), lambda i: (0, 0)),
            pl.BlockSpec((T, H), lambda i: (0, 0)),
            pl.BlockSpec((1, H), lambda i: (0, 0)),
            pl.BlockSpec((BI, C, T), lambda i: (i, 0, 0)),
            pl.BlockSpec((C, 1), lambda i: (0, 0)),
            pl.BlockSpec((1, 1), lambda i: (0, 0)),
        ],
        out_specs=pl.BlockSpec((BI, H), lambda i: (i, 0)),
        out_shape=jax.ShapeDtypeStruct((C, H), jnp.float32),
    )(cf, cf0, Wa, Wb, b_el.reshape(1, H), W1, W2, W3, W4, bk.reshape(1, H),
      e_logits, ex, has_any)


def _head_kernel(cfc_ref, wc_ref, bc_ref, ws_ref, bs_ref, w2_ref, b2_ref,
                 sem_ref, feat_ref):
    u = jnp.maximum(_dot(cfc_ref[...], wc_ref[...]) + bc_ref[...], 0.0)
    sem_ref[...] = _dot(u, ws_ref[...]) + bs_ref[...]
    feat_ref[...] = jnp.maximum(_dot(u, w2_ref[...]) + b2_ref[...], 0.0)


def _head(cf_cat, W_child, b_child, W_sem, b_sem, W_child2, b_child2):
    nsem = W_sem.shape[1]
    return pl.pallas_call(
        _head_kernel,
        out_shape=[
            jax.ShapeDtypeStruct((C, nsem), jnp.float32),
            jax.ShapeDtypeStruct((C, H), jnp.float32),
        ],
    )(cf_cat, W_child, b_child.reshape(1, H), W_sem, b_sem.reshape(1, nsem),
      W_child2, b_child2.reshape(1, H))


def kernel(parent_feature, gt_children_code, gt_num_code, W_parent, b_parent,
           W_exists, b_exists, W_sem, b_sem, W_edge_latent, b_edge_latent,
           W_edge_exists, b_edge_exists, W_node_edge, b_node_edge,
           W_child, b_child, W_child2, b_child2):
    feat = jnp.concatenate([parent_feature, gt_children_code, gt_num_code], axis=1)
    pf = _parent(feat, W_parent, b_parent)
    cf0 = pf.reshape(C, H)

    Wa = W_edge_latent[:H, :]
    Wb = W_edge_latent[H:, :]
    W_ee = W_edge_exists[:, :, 0]
    b_ee = b_edge_exists[:, 0]

    e_logits, ex, has_any = _edges(cf0, Wa, Wb, b_edge_latent, W_ee, b_ee,
                                   W_exists, b_exists)

    cf = cf0
    iterates = [cf0]
    for k in range(NITER):
        Wk = W_node_edge[k]
        cf = _mp_iter(cf, cf0, Wa, Wb, b_edge_latent,
                      Wk[:H, :], Wk[H:2 * H, :], Wk[2 * H:3 * H, :], Wk[3 * H:, :],
                      b_node_edge[k], e_logits, ex, has_any)
        iterates.append(cf)

    cf_cat = jnp.concatenate(iterates, axis=1)
    sem, feats = _head(cf_cat, W_child, b_child, W_sem, b_sem, W_child2, b_child2)

    return (feats.reshape(1, C, H),
            sem.reshape(1, C, W_sem.shape[1]),
            ex.reshape(1, C, 1),
            e_logits.reshape(1, C, C, T))
